# baseline (speedup 1.0000x reference)
import functools

import jax
import jax.numpy as jnp
from jax import lax
from jax.experimental import pallas as pl
from jax.experimental.pallas import tpu as pltpu
from jax.experimental.pallas import tpu_sc as plsc

_IGNORE = 255
_THRESH = 0.7
_MIN_KEPT = 100000

_NTILES = 32
_NPIXELS = 4 * 224 * 224
_CHUNK = _NPIXELS // _NTILES


def _ce_body(x_ref, t_ref, loss_ref, mp_ref):
    x = x_ref[0]
    t = t_ref[0]
    m = jnp.max(x, axis=0)
    sh = x - m[None]
    s = jnp.sum(jnp.exp(sh), axis=0)
    cls = lax.broadcasted_iota(jnp.int32, x.shape, 0)
    sh_t = jnp.sum(jnp.where(cls == t[None], sh, 0.0), axis=0)
    loss = jnp.log(s) - sh_t
    valid = t != _IGNORE
    loss_ref[0] = jnp.where(valid, loss, -1.0)
    mp_ref[0] = jnp.where(valid, 1.0 / s, 2.0)


def _sc_stats_body(loss_hbm, mp_hbm, stats_hbm, minv_hbm, lv, mv, sv, minsv):
    wid = lax.axis_index("s") * 2 + lax.axis_index("c")
    base = wid * _CHUNK
    pltpu.sync_copy(loss_hbm.at[pl.ds(base, _CHUNK)], lv)
    pltpu.sync_copy(mp_hbm.at[pl.ds(base, _CHUNK)], mv)

    def step(i, carry):
        acc_valid, acc_hard, acc_min = carry
        x = lv[pl.ds(i * 16, 16)]
        p = mv[pl.ds(i * 16, 16)]
        valid = x >= 0.0
        acc_valid = acc_valid + jnp.where(valid, 1, 0)
        acc_hard = acc_hard + jnp.where(p < _THRESH, 1, 0)
        acc_min = jnp.minimum(acc_min, jnp.where(valid, x, jnp.inf))
        return acc_valid, acc_hard, acc_min

    zero = jnp.zeros((16,), jnp.int32)
    acc_valid, acc_hard, acc_min = lax.fori_loop(
        0, _CHUNK // 16, step, (zero, zero, jnp.full((16,), jnp.inf, jnp.float32)))
    sv[0] = acc_valid
    sv[1] = acc_hard
    minsv[...] = acc_min
    pltpu.sync_copy(sv, stats_hbm.at[wid])
    pltpu.sync_copy(minsv, minv_hbm.at[wid])


_sc_stats = functools.partial(
    pl.kernel,
    out_type=[
        jax.ShapeDtypeStruct((_NTILES, 2, 16), jnp.int32),
        jax.ShapeDtypeStruct((_NTILES, 16), jnp.float32),
    ],
    mesh=plsc.VectorSubcoreMesh(core_axis_name="c", subcore_axis_name="s"),
    scratch_types=[
        pltpu.VMEM((_CHUNK,), jnp.float32),
        pltpu.VMEM((_CHUNK,), jnp.float32),
        pltpu.VMEM((2, 16), jnp.int32),
        pltpu.VMEM((16,), jnp.float32),
    ],
)(_sc_stats_body)


def _sel_body(loss_ref, stats_ref, minv_ref, out_ref):
    loss = loss_ref[...]
    stats = stats_ref[...]
    npix = jnp.sum(stats[:, 0, :])
    nhard = jnp.sum(stats[:, 1, :])
    min_kept = jnp.minimum(_MIN_KEPT, npix)
    k = jnp.minimum(jnp.maximum(min_kept, nhard), npix)
    bits = lax.bitcast_convert_type(loss, jnp.int32)

    def _fast(_):
        return jnp.min(minv_ref[...])

    def _slow(_):
        def step(_, lohi):
            lo, hi = lohi
            mid = lo + lax.div(hi - lo, 2)
            cnt = jnp.sum((bits >= mid).astype(jnp.int32))
            big = cnt >= k
            return jnp.where(big, mid, lo), jnp.where(big, hi, mid)

        lo, _ = lax.fori_loop(0, 31, step,
                              (jnp.int32(0), jnp.int32(0x7F800000)))
        return lax.bitcast_convert_type(lo, jnp.float32)

    thresh = lax.cond(k == npix, _fast, _slow, None)
    keep = loss >= thresh
    cnt = jnp.sum(keep.astype(jnp.int32))
    hsum = jnp.sum(jnp.where(keep, loss, 0.0))
    mean = hsum / jnp.maximum(cnt, 1).astype(jnp.float32)
    out_ref[...] = jnp.where(npix == 0, 0.0, mean).reshape(1, 1)


@functools.partial(jax.jit, static_argnames=("hb",))
def _run(logits, targets, hb=56):
    B, C, H, W = logits.shape
    nh = H // hb
    loss, mp = pl.pallas_call(
        _ce_body,
        grid=(B, nh),
        in_specs=[
            pl.BlockSpec((1, C, hb, W), lambda b, h: (b, 0, h, 0)),
            pl.BlockSpec((1, hb, W), lambda b, h: (b, h, 0)),
        ],
        out_specs=[
            pl.BlockSpec((1, hb, W), lambda b, h: (b, h, 0)),
            pl.BlockSpec((1, hb, W), lambda b, h: (b, h, 0)),
        ],
        out_shape=[
            jax.ShapeDtypeStruct((B, H, W), jnp.float32),
            jax.ShapeDtypeStruct((B, H, W), jnp.float32),
        ],
    )(logits, targets)
    stats, minv = _sc_stats(loss.reshape(-1), mp.reshape(-1))
    out = pl.pallas_call(
        _sel_body,
        out_shape=jax.ShapeDtypeStruct((1, 1), jnp.float32),
    )(loss, stats, minv)
    return out[0, 0]


def kernel(logits, targets):
    return _run(logits, targets)
